# baseline (speedup 1.0000x reference)
import jax
import jax.numpy as jnp
from jax.experimental import pallas as pl


_BM = 512


def _mlp_block(x_ref, w1_ref, b1_ref, w2_ref, b2_ref, o_ref):
    x = x_ref[...]
    h = jax.lax.dot_general(
        x, w1_ref[...], (((1,), (1,)), ((), ())),
        preferred_element_type=jnp.float32)
    h = jnp.maximum(h + b1_ref[...], 0.0)
    o = jax.lax.dot_general(
        h.astype(jnp.bfloat16), w2_ref[...], (((1,), (1,)), ((), ())),
        preferred_element_type=jnp.float32)
    o_ref[...] = o + b2_ref[...]


def kernel(input, W1, b1, W2, b2):
    M, K = input.shape
    N1, _ = W1.shape
    N2, _ = W2.shape
    xb = input.astype(jnp.bfloat16)
    w1b = W1.astype(jnp.bfloat16)
    w2b = W2.astype(jnp.bfloat16)
    b1r = b1.reshape(1, N1)
    b2r = b2.reshape(1, N2)
    return pl.pallas_call(
        _mlp_block,
        grid=(M // _BM,),
        in_specs=[
            pl.BlockSpec((_BM, K), lambda i: (i, 0)),
            pl.BlockSpec((N1, K), lambda i: (0, 0)),
            pl.BlockSpec((1, N1), lambda i: (0, 0)),
            pl.BlockSpec((N2, N1), lambda i: (0, 0)),
            pl.BlockSpec((1, N2), lambda i: (0, 0)),
        ],
        out_specs=pl.BlockSpec((_BM, N2), lambda i: (i, 0)),
        out_shape=jax.ShapeDtypeStruct((M, N2), jnp.float32),
    )(xb, w1b, b1r, w2b, b2r)


# speedup vs baseline: 1.3415x; 1.3415x over previous
import jax
import jax.numpy as jnp
from jax.experimental import pallas as pl


_BM = 512


def _mlp_block(x_ref, w1_ref, b1_ref, w2_ref, b2_ref, o_ref):
    x = x_ref[...]
    h = jax.lax.dot_general(
        x, w1_ref[...], (((1,), (1,)), ((), ())),
        preferred_element_type=jnp.float32)
    h = jnp.maximum(h + b1_ref[...], 0.0)
    o = jax.lax.dot_general(
        h.astype(jnp.bfloat16), w2_ref[...], (((1,), (1,)), ((), ())),
        preferred_element_type=jnp.float32)
    o_ref[...] = o + b2_ref[...]


def kernel(input, W1, b1, W2, b2):
    M, K = input.shape
    N1, _ = W1.shape
    N2, _ = W2.shape
    xb = input
    w1b = W1
    w2b = W2
    b1r = b1.reshape(1, N1)
    b2r = b2.reshape(1, N2)
    return pl.pallas_call(
        _mlp_block,
        grid=(M // _BM,),
        in_specs=[
            pl.BlockSpec((_BM, K), lambda i: (i, 0)),
            pl.BlockSpec((N1, K), lambda i: (0, 0)),
            pl.BlockSpec((1, N1), lambda i: (0, 0)),
            pl.BlockSpec((N2, N1), lambda i: (0, 0)),
            pl.BlockSpec((1, N2), lambda i: (0, 0)),
        ],
        out_specs=pl.BlockSpec((_BM, N2), lambda i: (i, 0)),
        out_shape=jax.ShapeDtypeStruct((M, N2), jnp.float32),
    )(xb, w1b, b1r, w2b, b2r)


# speedup vs baseline: 1.3666x; 1.0187x over previous
import jax
import jax.numpy as jnp
from jax.experimental import pallas as pl
from jax.experimental.pallas import tpu as pltpu


_BM = 512
_LAG = 2


def _mlp_block(x_ref, w1_ref, b1_ref, w2_hbm_ref, b2_ref, o_ref,
               h_scr, w2_vmem, dma_sem):
    i = pl.program_id(0)
    nsteps = pl.num_programs(0)
    w2_copy = pltpu.make_async_copy(w2_hbm_ref, w2_vmem, dma_sem)

    @pl.when(i == 0)
    def _start_w2():
        w2_copy.start()

    @pl.when(i < nsteps - _LAG)
    def _layer1():
        xb = x_ref[...].astype(jnp.bfloat16)
        h = jax.lax.dot_general(
            xb, w1_ref[...], (((1,), (1,)), ((), ())),
            preferred_element_type=jnp.float32)
        h = jnp.maximum(h + b1_ref[...], 0.0)
        h_scr[i % (_LAG + 1)] = h.astype(jnp.bfloat16)

    @pl.when(i == _LAG)
    def _wait_w2():
        w2_copy.wait()

    @pl.when(i >= _LAG)
    def _layer2():
        hb = h_scr[(i - _LAG) % (_LAG + 1)]
        o = jax.lax.dot_general(
            hb, w2_vmem[...], (((1,), (1,)), ((), ())),
            preferred_element_type=jnp.float32)
        o_ref[...] = o + b2_ref[...]


def kernel(input, W1, b1, W2, b2):
    M, K = input.shape
    N1, _ = W1.shape
    N2, _ = W2.shape
    nblocks = M // _BM
    grid = (nblocks + _LAG,)
    last = nblocks - 1
    return pl.pallas_call(
        _mlp_block,
        grid=grid,
        in_specs=[
            pl.BlockSpec((_BM, K), lambda i: (jnp.minimum(i, last), 0)),
            pl.BlockSpec((N1, K), lambda i: (0, 0)),
            pl.BlockSpec((1, N1), lambda i: (0, 0)),
            pl.BlockSpec(memory_space=pl.ANY),
            pl.BlockSpec((1, N2), lambda i: (0, 0)),
        ],
        out_specs=pl.BlockSpec((_BM, N2), lambda i: (jnp.maximum(i - _LAG, 0), 0)),
        out_shape=jax.ShapeDtypeStruct((M, N2), jnp.float32),
        scratch_shapes=[
            pltpu.VMEM((_LAG + 1, _BM, N1), jnp.bfloat16),
            pltpu.VMEM((N2, N1), jnp.float32),
            pltpu.SemaphoreType.DMA,
        ],
    )(input, W1, b1.reshape(1, N1), W2, b2.reshape(1, N2))
